# Initial kernel scaffold; baseline (speedup 1.0000x reference)
import jax
import jax.numpy as jnp
from jax.experimental import pallas as pl


def kernel(inputs, W, b):
    raise NotImplementedError("write your pallas kernel here")


# speedup vs baseline: 1.4085x; 1.4085x over previous
import functools

import jax
import jax.numpy as jnp
from jax import lax
from jax.experimental import pallas as pl
from jax.experimental.pallas import tpu as pltpu
from jax.experimental.pallas import tpu_sc as plsc

B, T, H, W_, C = 4, 512, 64, 64, 3
BT = B * T
PIX = H * W_
WORDS = PIX * C
BINS = 512
LOOKUP = 101
OUT = 128
PAD = (LOOKUP - 1) // 2
PW = 640

NW = 32
FRAMES_PER_W = BT // NW
STEPS = PIX // 16


def _make_hist_kernel():
    mesh = plsc.VectorSubcoreMesh(
        core_axis_name="c", subcore_axis_name="s", num_cores=2)

    @functools.partial(
        pl.kernel,
        out_type=jax.ShapeDtypeStruct((BT, BINS), jnp.int32),
        mesh=mesh,
        scratch_types=[
            pltpu.VMEM((WORDS,), jnp.int32),
            pltpu.VMEM((BINS,), jnp.int32),
            pltpu.SemaphoreType.DMA,
        ],
        compiler_params=pltpu.CompilerParams(needs_layout_passes=False),
    )
    def hist_kernel(pix_hbm, out_hbm, buf, hist, sem):
        wid = lax.axis_index("s") * 2 + lax.axis_index("c")
        base = wid * FRAMES_PER_W
        lane = lax.iota(jnp.int32, 16)
        idx_r = lane * 3
        ones = jnp.ones((16,), jnp.int32)
        zeros = jnp.zeros((16,), jnp.int32)

        def frame_body(f, carry):
            fr = base + f
            pltpu.async_copy(pix_hbm.at[fr], buf, sem).wait()

            def zero_body(i, c):
                hist[pl.ds(i * 16, 16)] = zeros
                return c

            lax.fori_loop(0, BINS // 16, zero_body, 0, unroll=8)

            def step_body(s, c):
                ir = idx_r + s * 48
                r = plsc.load_gather(buf, [ir])
                g = plsc.load_gather(buf, [ir + 1])
                bl = plsc.load_gather(buf, [ir + 2])
                binv = ((r >> 5) << 6) + ((g >> 5) << 3) + (bl >> 5)
                plsc.addupdate_scatter(hist, [binv], ones)
                return c

            lax.fori_loop(0, STEPS, step_body, 0, unroll=8)
            pltpu.sync_copy(hist, out_hbm.at[fr])
            return carry

        lax.fori_loop(0, FRAMES_PER_W, frame_body, 0)

    return hist_kernel


def _phase2_kernel(x_ref, w_ref, b_ref, o_ref, p_ref):
    x = x_ref[0].astype(jnp.float32)
    ss = jnp.sum(x * x, axis=1, keepdims=True)
    xn = x / jnp.maximum(jnp.sqrt(ss), 1e-12)
    sim = lax.dot_general(xn, xn, (((1,), (1,)), ((), ())),
                          preferred_element_type=jnp.float32)
    p_ref[:, :] = jnp.zeros((T, PW), jnp.float32)
    p_ref[:, PAD:PAD + T] = sim

    wmat = w_ref[...]
    bvec = b_ref[...]
    for blk in range(T // 128):
        t0 = blk * 128
        slab = p_ref[t0:t0 + 128, t0:t0 + 256]
        rows = lax.broadcasted_iota(jnp.int32, (128, 256), 0)
        for k in (1, 2, 4, 8, 16, 32, 64):
            rolled = jnp.concatenate([slab[:, k:], slab[:, :k]], axis=1)
            slab = jnp.where((rows & k) != 0, rolled, slab)
        band = slab[:, :LOOKUP]
        res = lax.dot_general(band, wmat, (((1,), (1,)), ((), ())),
                              preferred_element_type=jnp.float32)
        o_ref[0, t0:t0 + 128, :] = jnp.maximum(res + bvec, 0.0)


def _phase2(hist, wmat, bvec):
    x3 = hist.reshape(B, T, BINS)
    return pl.pallas_call(
        _phase2_kernel,
        out_shape=jax.ShapeDtypeStruct((B, T, OUT), jnp.float32),
        grid=(B,),
        in_specs=[
            pl.BlockSpec((1, T, BINS), lambda i: (i, 0, 0)),
            pl.BlockSpec((OUT, LOOKUP), lambda i: (0, 0)),
            pl.BlockSpec((1, OUT), lambda i: (0, 0)),
        ],
        out_specs=pl.BlockSpec((1, T, OUT), lambda i: (i, 0, 0)),
        scratch_shapes=[pltpu.VMEM((T, PW), jnp.float32)],
    )(x3, wmat, bvec.reshape(1, OUT))


@jax.jit
def kernel(inputs, W, b):
    pix = inputs.reshape(BT, WORDS)
    hist = _make_hist_kernel()(pix)
    return _phase2(hist, W, b)
